# baseline (speedup 1.0000x reference)
import dataclasses
import functools

import jax
import jax.numpy as jnp
from jax import lax
from jax.experimental import pallas as pl
from jax.experimental.pallas import tpu as pltpu
from jax.experimental.pallas import tpu_sc as plsc

N = 100000
INC = 128
OUTC = 128
K = 27
EK = 23000
E = K * EK

NC = 2
NS = 16
NW = NC * NS
CHUNK = 256
CHUNKS_PER_W = 76
E_PAD = NW * CHUNK * CHUNKS_PER_W

BLK = 1000
NBLK = E // BLK


def _sc_compiler_params(tc_tiling=True):
    cp = pltpu.CompilerParams()
    fields = pltpu.CompilerParams.__dataclass_fields__
    if "needs_layout_passes" in fields:
        cp = dataclasses.replace(cp, needs_layout_passes=False)
    if not tc_tiling and "use_tc_tiling_on_sc" in fields:
        cp = dataclasses.replace(cp, use_tc_tiling_on_sc=False)
    return cp


def _sc_gather(features, src_pad):
    mesh = plsc.VectorSubcoreMesh(core_axis_name="c", subcore_axis_name="s")

    @functools.partial(
        pl.kernel,
        out_type=jax.ShapeDtypeStruct((E_PAD, INC), jnp.float32),
        mesh=mesh,
        scratch_types=[
            pltpu.VMEM((CHUNK,), jnp.int32),
            pltpu.VMEM((CHUNK,), jnp.int32),
            pltpu.VMEM((CHUNK, INC), jnp.float32),
            pltpu.VMEM((CHUNK, INC), jnp.float32),
            pltpu.SemaphoreType.DMA,
            pltpu.SemaphoreType.DMA,
            pltpu.SemaphoreType.DMA,
            pltpu.SemaphoreType.DMA,
        ],
    )
    def k(feat_hbm, src_hbm, out_hbm, idx0, idx1, rows0, rows1,
          gsem0, gsem1, wsem0, wsem1):
        wid = lax.axis_index("s") * NC + lax.axis_index("c")
        idx = (idx0, idx1)
        rows = (rows0, rows1)
        gsem = (gsem0, gsem1)
        wsem = (wsem0, wsem1)
        gathers = [None, None]
        writes = [None, None]

        for j in range(CHUNKS_PER_W):
            p = j % 2
            base = (wid * CHUNKS_PER_W + j) * CHUNK
            if writes[p] is not None:
                writes[p].wait()
            pltpu.sync_copy(src_hbm.at[pl.ds(base, CHUNK)], idx[p])
            gathers[p] = pltpu.async_copy(feat_hbm.at[idx[p]], rows[p],
                                          gsem[p])
            q = 1 - p
            if gathers[q] is not None:
                gathers[q].wait()
                prev = base - CHUNK
                writes[q] = pltpu.async_copy(
                    rows[q], out_hbm.at[pl.ds(prev, CHUNK)], wsem[q])
                gathers[q] = None

        last = CHUNKS_PER_W - 1
        p = last % 2
        gathers[p].wait()
        base = (wid * CHUNKS_PER_W + last) * CHUNK
        pltpu.sync_copy(rows[p], out_hbm.at[pl.ds(base, CHUNK)])
        writes[1 - p].wait()

    return k(features, src_pad)


def _tc_matmul(gathered, weights):
    def body(g_ref, w_ref, t_ref):
        t_ref[...] = jnp.dot(g_ref[...], w_ref[0],
                             preferred_element_type=jnp.float32)

    return pl.pallas_call(
        body,
        grid=(NBLK,),
        in_specs=[
            pl.BlockSpec((BLK, INC), lambda i: (i, 0)),
            pl.BlockSpec((1, INC, OUTC), lambda i: (i // (EK // BLK), 0, 0)),
        ],
        out_specs=pl.BlockSpec((BLK, OUTC), lambda i: (i, 0)),
        out_shape=jax.ShapeDtypeStruct((E, OUTC), jnp.float32),
    )(gathered, weights)


NBKT = 25
BROWS = 4096
SHIFT = 12
NSLOT = 13
SLICE = E_PAD // NS
STRIP = 2048
NSTRIP = SLICE // STRIP
VPS = STRIP // 16
GRAN = 16
ROUND = 3 * GRAN
ARENA = -(-(SLICE + NSLOT * (ROUND - 1)) // ROUND) * ROUND
TRASH = BROWS
ACC_ROWS = BROWS + NS * 4


def _sc_scan(dst_pad):
    mesh = plsc.VectorSubcoreMesh(core_axis_name="c", subcore_axis_name="s")

    @functools.partial(
        pl.kernel,
        out_type=(
            jax.ShapeDtypeStruct((NC, NS, ARENA), jnp.int32),
            jax.ShapeDtypeStruct((NC, NS, ARENA), jnp.int32),
            jax.ShapeDtypeStruct((NC, NS, 32), jnp.int32),
        ),
        mesh=mesh,
        compiler_params=_sc_compiler_params(),
        scratch_types=[
            pltpu.VMEM((STRIP,), jnp.int32),
            pltpu.VMEM((ARENA,), jnp.int32),
            pltpu.VMEM((ARENA,), jnp.int32),
            pltpu.VMEM((32,), jnp.int32),
        ],
    )
    def k(dst_hbm, e_out, loc_out, meta_out, dstb_v, e_ar, loc_ar, meta_v):
        c = lax.axis_index("c")
        t = lax.axis_index("s")
        iota = lax.iota(jnp.int32, 16)
        base = t * SLICE

        def strip1(s_i, cnts):
            pltpu.sync_copy(dst_hbm.at[pl.ds(base + s_i * STRIP, STRIP)],
                            dstb_v)

            def vec1(v, cnts):
                d = dstb_v[pl.ds(v * 16, 16)]
                bkt = d >> SHIFT
                return tuple(
                    cnts[s] + jnp.sum(jnp.where(bkt == 2 * s + c, 1, 0))
                    for s in range(NSLOT))

            return lax.fori_loop(0, VPS, vec1, cnts)

        cnts = lax.fori_loop(0, NSTRIP, strip1,
                             tuple(jnp.int32(0) for _ in range(NSLOT)))

        offs = []
        run = jnp.int32(0)
        meta_lo = jnp.zeros((16,), jnp.int32)
        for s in range(NSLOT):
            offs.append(run)
            meta_lo = jnp.where(iota == s, run, meta_lo)
            run = run + ((cnts[s] + ROUND - 1) // ROUND) * ROUND

        def strip2(s_i, curs):
            pltpu.sync_copy(dst_hbm.at[pl.ds(base + s_i * STRIP, STRIP)],
                            dstb_v)

            def vec2(v, curs):
                d = dstb_v[pl.ds(v * 16, 16)]
                bkt = d >> SHIFT
                e_vec = base + s_i * STRIP + v * 16 + iota
                loc_vec = jnp.bitwise_and(d, BROWS - 1)
                new = []
                for s in range(NSLOT):
                    m = bkt == 2 * s + c
                    mi = jnp.where(m, 1, 0)
                    r = plsc.cumsum(mi)
                    pos = curs[s] + r - 1
                    plsc.store_scatter(e_ar, [pos], e_vec, mask=m)
                    plsc.store_scatter(loc_ar, [pos], loc_vec, mask=m)
                    new.append(curs[s] + jnp.sum(mi))
                return tuple(new)

            return lax.fori_loop(0, VPS, vec2, curs)

        curs = lax.fori_loop(0, NSTRIP, strip2, tuple(offs))

        trash = TRASH + t * 4 + jnp.bitwise_and(iota, 3)
        meta_hi = jnp.zeros((16,), jnp.int32)
        for s in range(NSLOT):
            cnt = curs[s] - offs[s]
            end = offs[s] + ((cnt + ROUND - 1) // ROUND) * ROUND
            for kq in range(ROUND // 16):
                pos = curs[s] + kq * 16 + iota
                m = pos < end
                plsc.store_scatter(e_ar, [pos], (kq % 4) * 16 + iota, mask=m)
                plsc.store_scatter(loc_ar, [pos], trash, mask=m)
            meta_hi = jnp.where(iota == s, end, meta_hi)

        meta_v[pl.ds(0, 16)] = meta_lo
        meta_v[pl.ds(16, 16)] = meta_hi
        pltpu.sync_copy(e_ar, e_out.at[c, t])
        pltpu.sync_copy(loc_ar, loc_out.at[c, t])
        pltpu.sync_copy(meta_v, meta_out.at[c, t])

    return k(dst_pad)


def _sc_scatter_add(transformed, e_out, loc_out, meta_out):
    mesh = plsc.VectorSubcoreMesh(core_axis_name="c", subcore_axis_name="s")
    NRING = 3

    @functools.partial(
        pl.kernel,
        out_type=jax.ShapeDtypeStruct((N, OUTC), jnp.float32),
        mesh=mesh,
        compiler_params=_sc_compiler_params(tc_tiling=False),
        scratch_types=(
            [pltpu.VMEM((32,), jnp.int32),
             pltpu.VMEM((8, OUTC), jnp.float32),
             pltpu.VMEM_SHARED((ACC_ROWS, OUTC), jnp.float32)]
            + [pltpu.VMEM((GRAN,), jnp.int32)] * NRING
            + [pltpu.VMEM((GRAN,), jnp.int32)] * NRING
            + [pltpu.VMEM((GRAN, OUTC), jnp.float32)] * NRING
            + [pltpu.SemaphoreType.DMA] * (2 * NRING)
        ),
    )
    def k(t_hbm, e_hbm, loc_hbm, meta_hbm, out_hbm, meta_v, zero_v, acc,
          *ring):
        ering = ring[0:NRING]
        lring = ring[NRING:2 * NRING]
        buf = ring[2 * NRING:3 * NRING]
        isem = ring[3 * NRING:3 * NRING + NRING]
        gsem = ring[3 * NRING + NRING:]
        c = lax.axis_index("c")
        t = lax.axis_index("s")
        iota = lax.iota(jnp.int32, 16)
        slab = BROWS // NS

        pltpu.sync_copy(meta_hbm.at[c, t], meta_v)
        for r in range(8):
            for q in range(OUTC // 16):
                zero_v[r, pl.ds(q * 16, 16)] = jnp.zeros((16,), jnp.float32)
        meta_lo = meta_v[pl.ds(0, 16)]
        meta_hi = meta_v[pl.ds(16, 16)]

        def idx_load(g, p):
            pltpu.async_copy(e_hbm.at[c, t, pl.ds(g * GRAN, GRAN)],
                             ering[p], isem[p])
            pltpu.async_copy(loc_hbm.at[c, t, pl.ds(g * GRAN, GRAN)],
                             lring[p], isem[p])

        def idx_wait(p):
            pltpu.make_async_copy(e_hbm.at[0, 0, pl.ds(0, GRAN)],
                                  ering[p], isem[p]).wait()
            pltpu.make_async_copy(loc_hbm.at[0, 0, pl.ds(0, GRAN)],
                                  lring[p], isem[p]).wait()

        def accumulate(s):
            @pl.loop(0, slab // 8)
            def _(i):
                pltpu.sync_copy(zero_v, acc.at[pl.ds(t * slab + i * 8, 8)])

            plsc.subcore_barrier()
            g0 = jnp.sum(jnp.where(iota == s, meta_lo, 0)) >> 4
            g1 = jnp.sum(jnp.where(iota == s, meta_hi, 0)) >> 4

            for p in range(NRING):
                @pl.when(g0 + p < g1)
                def _():
                    idx_load(g0 + p, p)

            def round_body(rr, _):
                g = g0 + rr * NRING
                for p in range(NRING):
                    idx_wait(p)
                    pltpu.async_copy(t_hbm.at[ering[p]], buf[p], gsem[p])
                for p in range(NRING):
                    pltpu.make_async_copy(t_hbm.at[ering[p]], buf[p],
                                          gsem[p]).wait()
                    pltpu.sync_copy(buf[p], acc.at[lring[p]], add=True)
                    nxt = g + p + NRING

                    @pl.when(nxt < g1)
                    def _():
                        idx_load(nxt, p)
                return 0

            lax.fori_loop(0, (g1 - g0) // NRING, round_body, 0)
            plsc.subcore_barrier()

        def bucket_body(s, _):
            accumulate(s)
            b = 2 * s + c
            pltpu.sync_copy(
                acc.at[pl.ds(t * slab, slab)],
                out_hbm.at[pl.ds(b * BROWS + t * slab, slab)])
            return 0

        lax.fori_loop(0, (NBKT - 1) // 2, bucket_body, 0)

        tail_rows = N - (NBKT - 1) * BROWS
        full = tail_rows // slab
        rem = tail_rows % slab

        @pl.when(c == (NBKT - 1) % 2)
        def _():
            accumulate(jnp.int32((NBKT - 1) // 2))

            @pl.when(t < full)
            def _():
                pltpu.sync_copy(
                    acc.at[pl.ds(t * slab, slab)],
                    out_hbm.at[pl.ds((NBKT - 1) * BROWS + t * slab, slab)])

            if rem:

                @pl.when(t == full)
                def _():
                    pltpu.sync_copy(
                        acc.at[pl.ds(full * slab, rem)],
                        out_hbm.at[
                            pl.ds((NBKT - 1) * BROWS + full * slab, rem)])

    return k(transformed, e_out, loc_out, meta_out)


def kernel(features, nbmap, coords, kernel):
    src = nbmap[:, :, 0].reshape(-1)
    src_pad = jnp.concatenate([src, jnp.zeros((E_PAD - E,), jnp.int32)])
    dst = nbmap[:, :, 1].reshape(-1)
    dst_pad = jnp.concatenate(
        [dst, jnp.full((E_PAD - E,), 1 << 20, jnp.int32)])
    e_out, loc_out, meta_out = _sc_scan(dst_pad)
    gathered = _sc_gather(features, src_pad)
    transformed = _tc_matmul(gathered, kernel)
    return _sc_scatter_add(transformed, e_out, loc_out, meta_out)


# speedup vs baseline: 1.0518x; 1.0518x over previous
import dataclasses
import functools

import jax
import jax.numpy as jnp
from jax import lax
from jax.experimental import pallas as pl
from jax.experimental.pallas import tpu as pltpu
from jax.experimental.pallas import tpu_sc as plsc

N = 100000
INC = 128
OUTC = 128
K = 27
EK = 23000
E = K * EK

NC = 2
NS = 16
NW = NC * NS
CHUNK = 256
CHUNKS_PER_W = 76
E_PAD = NW * CHUNK * CHUNKS_PER_W

BLK = 1000
NBLK = E // BLK


def _sc_compiler_params(tc_tiling=True):
    cp = pltpu.CompilerParams()
    fields = pltpu.CompilerParams.__dataclass_fields__
    if "needs_layout_passes" in fields:
        cp = dataclasses.replace(cp, needs_layout_passes=False)
    if not tc_tiling and "use_tc_tiling_on_sc" in fields:
        cp = dataclasses.replace(cp, use_tc_tiling_on_sc=False)
    return cp


def _sc_gather(features, src_pad):
    mesh = plsc.VectorSubcoreMesh(core_axis_name="c", subcore_axis_name="s")

    @functools.partial(
        pl.kernel,
        out_type=jax.ShapeDtypeStruct((E_PAD, INC), jnp.float32),
        mesh=mesh,
        scratch_types=[
            pltpu.VMEM((CHUNK,), jnp.int32),
            pltpu.VMEM((CHUNK,), jnp.int32),
            pltpu.VMEM((CHUNK, INC), jnp.float32),
            pltpu.VMEM((CHUNK, INC), jnp.float32),
            pltpu.SemaphoreType.DMA,
            pltpu.SemaphoreType.DMA,
            pltpu.SemaphoreType.DMA,
            pltpu.SemaphoreType.DMA,
        ],
    )
    def k(feat_hbm, src_hbm, out_hbm, idx0, idx1, rows0, rows1,
          gsem0, gsem1, wsem0, wsem1):
        wid = lax.axis_index("s") * NC + lax.axis_index("c")
        idx = (idx0, idx1)
        rows = (rows0, rows1)
        gsem = (gsem0, gsem1)
        wsem = (wsem0, wsem1)
        gathers = [None, None]
        writes = [None, None]

        for j in range(CHUNKS_PER_W):
            p = j % 2
            base = (wid * CHUNKS_PER_W + j) * CHUNK
            if writes[p] is not None:
                writes[p].wait()
            pltpu.sync_copy(src_hbm.at[pl.ds(base, CHUNK)], idx[p])
            gathers[p] = pltpu.async_copy(feat_hbm.at[idx[p]], rows[p],
                                          gsem[p])
            q = 1 - p
            if gathers[q] is not None:
                gathers[q].wait()
                prev = base - CHUNK
                writes[q] = pltpu.async_copy(
                    rows[q], out_hbm.at[pl.ds(prev, CHUNK)], wsem[q])
                gathers[q] = None

        last = CHUNKS_PER_W - 1
        p = last % 2
        gathers[p].wait()
        base = (wid * CHUNKS_PER_W + last) * CHUNK
        pltpu.sync_copy(rows[p], out_hbm.at[pl.ds(base, CHUNK)])
        writes[1 - p].wait()

    return k(features, src_pad)


def _tc_matmul(gathered, weights):
    def body(g_ref, w_ref, t_ref):
        t_ref[...] = jnp.dot(g_ref[...], w_ref[0],
                             preferred_element_type=jnp.float32)

    return pl.pallas_call(
        body,
        grid=(NBLK,),
        in_specs=[
            pl.BlockSpec((BLK, INC), lambda i: (i, 0)),
            pl.BlockSpec((1, INC, OUTC), lambda i: (i // (EK // BLK), 0, 0)),
        ],
        out_specs=pl.BlockSpec((BLK, OUTC), lambda i: (i, 0)),
        out_shape=jax.ShapeDtypeStruct((E, OUTC), jnp.float32),
    )(gathered, weights)


NBKT = 25
BROWS = 4096
SHIFT = 12
NSLOT = 13
SLICE = E_PAD // NS
STRIP = 2048
NSTRIP = SLICE // STRIP
VPS = STRIP // 16
GRAN = 16
ROUND = 3 * GRAN
ARENA = -(-(SLICE + NSLOT * (ROUND - 1)) // ROUND) * ROUND
TRASH = BROWS
ACC_ROWS = BROWS + NS * 4


def _sc_scan(dst_pad):
    mesh = plsc.VectorSubcoreMesh(core_axis_name="c", subcore_axis_name="s")

    @functools.partial(
        pl.kernel,
        out_type=(
            jax.ShapeDtypeStruct((NC, NS, ARENA), jnp.int32),
            jax.ShapeDtypeStruct((NC, NS, ARENA), jnp.int32),
            jax.ShapeDtypeStruct((NC, NS, 32), jnp.int32),
        ),
        mesh=mesh,
        compiler_params=_sc_compiler_params(),
        scratch_types=[
            pltpu.VMEM((STRIP,), jnp.int32),
            pltpu.VMEM((ARENA,), jnp.int32),
            pltpu.VMEM((ARENA,), jnp.int32),
            pltpu.VMEM((32,), jnp.int32),
        ],
    )
    def k(dst_hbm, e_out, loc_out, meta_out, dstb_v, e_ar, loc_ar, meta_v):
        c = lax.axis_index("c")
        t = lax.axis_index("s")
        iota = lax.iota(jnp.int32, 16)
        base = t * SLICE

        def strip1(s_i, cnts):
            pltpu.sync_copy(dst_hbm.at[pl.ds(base + s_i * STRIP, STRIP)],
                            dstb_v)

            def vec1(v, cnts):
                d = dstb_v[pl.ds(v * 16, 16)]
                bkt = d >> SHIFT
                return tuple(
                    cnts[s] + jnp.sum(jnp.where(bkt == 2 * s + c, 1, 0))
                    for s in range(NSLOT))

            return lax.fori_loop(0, VPS, vec1, cnts)

        cnts = lax.fori_loop(0, NSTRIP, strip1,
                             tuple(jnp.int32(0) for _ in range(NSLOT)))

        offs = []
        run = jnp.int32(0)
        meta_lo = jnp.zeros((16,), jnp.int32)
        for s in range(NSLOT):
            offs.append(run)
            meta_lo = jnp.where(iota == s, run, meta_lo)
            run = run + ((cnts[s] + ROUND - 1) // ROUND) * ROUND

        def strip2(s_i, curs):
            pltpu.sync_copy(dst_hbm.at[pl.ds(base + s_i * STRIP, STRIP)],
                            dstb_v)

            def vec2(v, curs):
                d = dstb_v[pl.ds(v * 16, 16)]
                bkt = d >> SHIFT
                e_vec = base + s_i * STRIP + v * 16 + iota
                loc_vec = jnp.bitwise_and(d, BROWS - 1)
                new = []
                for s in range(NSLOT):
                    m = bkt == 2 * s + c
                    mi = jnp.where(m, 1, 0)
                    r = plsc.cumsum(mi)
                    pos = curs[s] + r - 1
                    plsc.store_scatter(e_ar, [pos], e_vec, mask=m)
                    plsc.store_scatter(loc_ar, [pos], loc_vec, mask=m)
                    new.append(curs[s] + jnp.sum(mi))
                return tuple(new)

            return lax.fori_loop(0, VPS, vec2, curs)

        curs = lax.fori_loop(0, NSTRIP, strip2, tuple(offs))

        trash = TRASH + t * 4 + jnp.bitwise_and(iota, 3)
        meta_hi = jnp.zeros((16,), jnp.int32)
        for s in range(NSLOT):
            cnt = curs[s] - offs[s]
            end = offs[s] + ((cnt + ROUND - 1) // ROUND) * ROUND
            for kq in range(ROUND // 16):
                pos = curs[s] + kq * 16 + iota
                m = pos < end
                plsc.store_scatter(e_ar, [pos], (kq % 4) * 16 + iota, mask=m)
                plsc.store_scatter(loc_ar, [pos], trash, mask=m)
            meta_hi = jnp.where(iota == s, end, meta_hi)

        meta_v[pl.ds(0, 16)] = meta_lo
        meta_v[pl.ds(16, 16)] = meta_hi
        pltpu.sync_copy(e_ar, e_out.at[c, t])
        pltpu.sync_copy(loc_ar, loc_out.at[c, t])
        pltpu.sync_copy(meta_v, meta_out.at[c, t])

    return k(dst_pad)


def _sc_scatter_add(transformed, e_out, loc_out, meta_out):
    mesh = plsc.VectorSubcoreMesh(core_axis_name="c", subcore_axis_name="s")
    NRING = 3

    @functools.partial(
        pl.kernel,
        out_type=jax.ShapeDtypeStruct((N, OUTC), jnp.float32),
        mesh=mesh,
        compiler_params=_sc_compiler_params(tc_tiling=False),
        scratch_types=(
            [pltpu.VMEM((32,), jnp.int32),
             pltpu.VMEM((8, OUTC), jnp.float32),
             pltpu.VMEM_SHARED((ACC_ROWS, OUTC), jnp.float32)]
            + [pltpu.VMEM((GRAN,), jnp.int32)] * NRING
            + [pltpu.VMEM((GRAN,), jnp.int32)] * NRING
            + [pltpu.VMEM((GRAN, OUTC), jnp.float32)] * NRING
            + [pltpu.SemaphoreType.DMA] * (3 * NRING)
        ),
    )
    def k(t_hbm, e_hbm, loc_hbm, meta_hbm, out_hbm, meta_v, zero_v, acc,
          *ring):
        ering = ring[0:NRING]
        lring = ring[NRING:2 * NRING]
        buf = ring[2 * NRING:3 * NRING]
        isem = ring[3 * NRING:4 * NRING]
        gsem = ring[4 * NRING:5 * NRING]
        asem = ring[5 * NRING:6 * NRING]
        c = lax.axis_index("c")
        t = lax.axis_index("s")
        iota = lax.iota(jnp.int32, 16)
        slab = BROWS // NS

        pltpu.sync_copy(meta_hbm.at[c, t], meta_v)
        for r in range(8):
            for q in range(OUTC // 16):
                zero_v[r, pl.ds(q * 16, 16)] = jnp.zeros((16,), jnp.float32)
        meta_lo = meta_v[pl.ds(0, 16)]
        meta_hi = meta_v[pl.ds(16, 16)]

        def idx_load(g, p):
            pltpu.async_copy(e_hbm.at[c, t, pl.ds(g * GRAN, GRAN)],
                             ering[p], isem[p])
            pltpu.async_copy(loc_hbm.at[c, t, pl.ds(g * GRAN, GRAN)],
                             lring[p], isem[p])

        def idx_wait(p):
            pltpu.make_async_copy(e_hbm.at[0, 0, pl.ds(0, GRAN)],
                                  ering[p], isem[p]).wait()
            pltpu.make_async_copy(loc_hbm.at[0, 0, pl.ds(0, GRAN)],
                                  lring[p], isem[p]).wait()

        def accumulate(s):
            zcopies = [
                pltpu.async_copy(zero_v, acc.at[pl.ds(t * slab + i * 8, 8)],
                                 gsem[i % NRING])
                for i in range(slab // 8)]
            for cp in zcopies:
                cp.wait()

            plsc.subcore_barrier()
            g0 = jnp.sum(jnp.where(iota == s, meta_lo, 0)) >> 4
            g1 = jnp.sum(jnp.where(iota == s, meta_hi, 0)) >> 4

            for p in range(NRING):
                @pl.when(g0 + p < g1)
                def _():
                    idx_load(g0 + p, p)

            def add_wait(p):
                pltpu.make_async_copy(buf[p], acc.at[lring[p]],
                                      asem[p]).wait()

            def round_body(rr, _):
                g = g0 + rr * NRING
                for p in range(NRING):
                    idx_wait(p)

                    @pl.when(rr > 0)
                    def _():
                        add_wait(p)

                    pltpu.async_copy(t_hbm.at[ering[p]], buf[p], gsem[p])
                for p in range(NRING):
                    pltpu.make_async_copy(t_hbm.at[ering[p]], buf[p],
                                          gsem[p]).wait()
                    pltpu.async_copy(buf[p], acc.at[lring[p]], asem[p],
                                     add=True)
                    nxt = g + p + NRING

                    @pl.when(nxt < g1)
                    def _():
                        idx_load(nxt, p)
                return 0

            rounds = (g1 - g0) // NRING
            lax.fori_loop(0, rounds, round_body, 0)
            for p in range(NRING):
                @pl.when(rounds > 0)
                def _():
                    add_wait(p)
            plsc.subcore_barrier()

        def bucket_body(s, _):
            accumulate(s)
            b = 2 * s + c
            pltpu.sync_copy(
                acc.at[pl.ds(t * slab, slab)],
                out_hbm.at[pl.ds(b * BROWS + t * slab, slab)])
            return 0

        lax.fori_loop(0, (NBKT - 1) // 2, bucket_body, 0)

        tail_rows = N - (NBKT - 1) * BROWS
        full = tail_rows // slab
        rem = tail_rows % slab

        @pl.when(c == (NBKT - 1) % 2)
        def _():
            accumulate(jnp.int32((NBKT - 1) // 2))

            @pl.when(t < full)
            def _():
                pltpu.sync_copy(
                    acc.at[pl.ds(t * slab, slab)],
                    out_hbm.at[pl.ds((NBKT - 1) * BROWS + t * slab, slab)])

            if rem:

                @pl.when(t == full)
                def _():
                    pltpu.sync_copy(
                        acc.at[pl.ds(full * slab, rem)],
                        out_hbm.at[
                            pl.ds((NBKT - 1) * BROWS + full * slab, rem)])

    return k(transformed, e_out, loc_out, meta_out)


def kernel(features, nbmap, coords, kernel):
    src = nbmap[:, :, 0].reshape(-1)
    src_pad = jnp.concatenate([src, jnp.zeros((E_PAD - E,), jnp.int32)])
    dst = nbmap[:, :, 1].reshape(-1)
    dst_pad = jnp.concatenate(
        [dst, jnp.full((E_PAD - E,), 1 << 20, jnp.int32)])
    e_out, loc_out, meta_out = _sc_scan(dst_pad)
    gathered = _sc_gather(features, src_pad)
    transformed = _tc_matmul(gathered, kernel)
    return _sc_scatter_add(transformed, e_out, loc_out, meta_out)


# speedup vs baseline: 1.2757x; 1.2129x over previous
import dataclasses
import functools

import jax
import jax.numpy as jnp
from jax import lax
from jax.experimental import pallas as pl
from jax.experimental.pallas import tpu as pltpu
from jax.experimental.pallas import tpu_sc as plsc

N = 100000
INC = 128
OUTC = 128
K = 27
EK = 23000
E = K * EK

NC = 2
NS = 16
NW = NC * NS
CHUNK = 256
CHUNKS_PER_W = 76
E_PAD = NW * CHUNK * CHUNKS_PER_W

BLK = 4600
NBLK = E // BLK


def _sc_compiler_params(tc_tiling=True):
    cp = pltpu.CompilerParams()
    fields = pltpu.CompilerParams.__dataclass_fields__
    if "needs_layout_passes" in fields:
        cp = dataclasses.replace(cp, needs_layout_passes=False)
    if not tc_tiling and "use_tc_tiling_on_sc" in fields:
        cp = dataclasses.replace(cp, use_tc_tiling_on_sc=False)
    return cp


def _sc_gather(features, src_pad):
    mesh = plsc.VectorSubcoreMesh(core_axis_name="c", subcore_axis_name="s")

    @functools.partial(
        pl.kernel,
        out_type=jax.ShapeDtypeStruct((E_PAD, INC), jnp.float32),
        mesh=mesh,
        scratch_types=[
            pltpu.VMEM((CHUNK,), jnp.int32),
            pltpu.VMEM((CHUNK,), jnp.int32),
            pltpu.VMEM((CHUNK, INC), jnp.float32),
            pltpu.VMEM((CHUNK, INC), jnp.float32),
            pltpu.SemaphoreType.DMA,
            pltpu.SemaphoreType.DMA,
            pltpu.SemaphoreType.DMA,
            pltpu.SemaphoreType.DMA,
        ],
    )
    def k(feat_hbm, src_hbm, out_hbm, idx0, idx1, rows0, rows1,
          gsem0, gsem1, wsem0, wsem1):
        wid = lax.axis_index("s") * NC + lax.axis_index("c")
        idx = (idx0, idx1)
        rows = (rows0, rows1)
        gsem = (gsem0, gsem1)
        wsem = (wsem0, wsem1)
        gathers = [None, None]
        writes = [None, None]

        for j in range(CHUNKS_PER_W):
            p = j % 2
            base = (wid * CHUNKS_PER_W + j) * CHUNK
            if writes[p] is not None:
                writes[p].wait()
            pltpu.sync_copy(src_hbm.at[pl.ds(base, CHUNK)], idx[p])
            gathers[p] = pltpu.async_copy(feat_hbm.at[idx[p]], rows[p],
                                          gsem[p])
            q = 1 - p
            if gathers[q] is not None:
                gathers[q].wait()
                prev = base - CHUNK
                writes[q] = pltpu.async_copy(
                    rows[q], out_hbm.at[pl.ds(prev, CHUNK)], wsem[q])
                gathers[q] = None

        last = CHUNKS_PER_W - 1
        p = last % 2
        gathers[p].wait()
        base = (wid * CHUNKS_PER_W + last) * CHUNK
        pltpu.sync_copy(rows[p], out_hbm.at[pl.ds(base, CHUNK)])
        writes[1 - p].wait()

    return k(features, src_pad)


def _tc_matmul(gathered, weights):
    def body(g_ref, w_ref, t_ref):
        t_ref[...] = jnp.dot(g_ref[...], w_ref[0],
                             preferred_element_type=jnp.float32)

    return pl.pallas_call(
        body,
        grid=(NBLK,),
        in_specs=[
            pl.BlockSpec((BLK, INC), lambda i: (i, 0)),
            pl.BlockSpec((1, INC, OUTC), lambda i: (i // (EK // BLK), 0, 0)),
        ],
        out_specs=pl.BlockSpec((BLK, OUTC), lambda i: (i, 0)),
        out_shape=jax.ShapeDtypeStruct((E, OUTC), jnp.float32),
    )(gathered, weights)


NBKT = 25
BROWS = 4096
SHIFT = 12
NSLOT = 13
SLICE = E_PAD // NS
STRIP = 2048
NSTRIP = SLICE // STRIP
VPS = STRIP // 16
GRAN = 16
ROUND = 3 * GRAN
ARENA = -(-(SLICE + NSLOT * (ROUND - 1)) // ROUND) * ROUND
TRASH = BROWS
ACC_ROWS = BROWS + NS * 4


def _sc_scan(dst_pad):
    mesh = plsc.VectorSubcoreMesh(core_axis_name="c", subcore_axis_name="s")

    @functools.partial(
        pl.kernel,
        out_type=(
            jax.ShapeDtypeStruct((NC, NS, ARENA), jnp.int32),
            jax.ShapeDtypeStruct((NC, NS, ARENA), jnp.int32),
            jax.ShapeDtypeStruct((NC, NS, 32), jnp.int32),
        ),
        mesh=mesh,
        compiler_params=_sc_compiler_params(),
        scratch_types=[
            pltpu.VMEM((STRIP,), jnp.int32),
            pltpu.VMEM((ARENA,), jnp.int32),
            pltpu.VMEM((ARENA,), jnp.int32),
            pltpu.VMEM((32,), jnp.int32),
        ],
    )
    def k(dst_hbm, e_out, loc_out, meta_out, dstb_v, e_ar, loc_ar, meta_v):
        c = lax.axis_index("c")
        t = lax.axis_index("s")
        iota = lax.iota(jnp.int32, 16)
        base = t * SLICE

        def strip1(s_i, cnts):
            pltpu.sync_copy(dst_hbm.at[pl.ds(base + s_i * STRIP, STRIP)],
                            dstb_v)

            def vec1(v, cnts):
                d = dstb_v[pl.ds(v * 16, 16)]
                bkt = d >> SHIFT
                return tuple(
                    cnts[s] + jnp.sum(jnp.where(bkt == 2 * s + c, 1, 0))
                    for s in range(NSLOT))

            return lax.fori_loop(0, VPS, vec1, cnts)

        cnts = lax.fori_loop(0, NSTRIP, strip1,
                             tuple(jnp.int32(0) for _ in range(NSLOT)))

        offs = []
        run = jnp.int32(0)
        meta_lo = jnp.zeros((16,), jnp.int32)
        for s in range(NSLOT):
            offs.append(run)
            meta_lo = jnp.where(iota == s, run, meta_lo)
            run = run + ((cnts[s] + ROUND - 1) // ROUND) * ROUND

        def strip2(s_i, curs):
            pltpu.sync_copy(dst_hbm.at[pl.ds(base + s_i * STRIP, STRIP)],
                            dstb_v)

            def vec2(v, curs):
                d = dstb_v[pl.ds(v * 16, 16)]
                bkt = d >> SHIFT
                e_vec = base + s_i * STRIP + v * 16 + iota
                loc_vec = jnp.bitwise_and(d, BROWS - 1)
                new = []
                for s in range(NSLOT):
                    m = bkt == 2 * s + c
                    mi = jnp.where(m, 1, 0)
                    r = plsc.cumsum(mi)
                    pos = curs[s] + r - 1
                    plsc.store_scatter(e_ar, [pos], e_vec, mask=m)
                    plsc.store_scatter(loc_ar, [pos], loc_vec, mask=m)
                    new.append(curs[s] + jnp.sum(mi))
                return tuple(new)

            return lax.fori_loop(0, VPS, vec2, curs)

        curs = lax.fori_loop(0, NSTRIP, strip2, tuple(offs))

        trash = TRASH + t * 4 + jnp.bitwise_and(iota, 3)
        meta_hi = jnp.zeros((16,), jnp.int32)
        for s in range(NSLOT):
            cnt = curs[s] - offs[s]
            end = offs[s] + ((cnt + ROUND - 1) // ROUND) * ROUND
            for kq in range(ROUND // 16):
                pos = curs[s] + kq * 16 + iota
                m = pos < end
                plsc.store_scatter(e_ar, [pos], (kq % 4) * 16 + iota, mask=m)
                plsc.store_scatter(loc_ar, [pos], trash, mask=m)
            meta_hi = jnp.where(iota == s, end, meta_hi)

        meta_v[pl.ds(0, 16)] = meta_lo
        meta_v[pl.ds(16, 16)] = meta_hi
        pltpu.sync_copy(e_ar, e_out.at[c, t])
        pltpu.sync_copy(loc_ar, loc_out.at[c, t])
        pltpu.sync_copy(meta_v, meta_out.at[c, t])

    return k(dst_pad)


def _sc_scatter_add(transformed, e_out, loc_out, meta_out):
    mesh = plsc.VectorSubcoreMesh(core_axis_name="c", subcore_axis_name="s")
    NRING = 3

    @functools.partial(
        pl.kernel,
        out_type=jax.ShapeDtypeStruct((N, OUTC), jnp.float32),
        mesh=mesh,
        compiler_params=_sc_compiler_params(tc_tiling=False),
        scratch_types=(
            [pltpu.VMEM((32,), jnp.int32),
             pltpu.VMEM((8, OUTC), jnp.float32),
             pltpu.VMEM_SHARED((ACC_ROWS, OUTC), jnp.float32)]
            + [pltpu.VMEM((GRAN,), jnp.int32)] * NRING
            + [pltpu.VMEM((GRAN,), jnp.int32)] * NRING
            + [pltpu.VMEM((GRAN, OUTC), jnp.float32)] * NRING
            + [pltpu.SemaphoreType.DMA] * (3 * NRING)
        ),
    )
    def k(t_hbm, e_hbm, loc_hbm, meta_hbm, out_hbm, meta_v, zero_v, acc,
          *ring):
        ering = ring[0:NRING]
        lring = ring[NRING:2 * NRING]
        buf = ring[2 * NRING:3 * NRING]
        isem = ring[3 * NRING:4 * NRING]
        gsem = ring[4 * NRING:5 * NRING]
        asem = ring[5 * NRING:6 * NRING]
        c = lax.axis_index("c")
        t = lax.axis_index("s")
        iota = lax.iota(jnp.int32, 16)
        slab = BROWS // NS

        pltpu.sync_copy(meta_hbm.at[c, t], meta_v)
        for r in range(8):
            for q in range(OUTC // 16):
                zero_v[r, pl.ds(q * 16, 16)] = jnp.zeros((16,), jnp.float32)
        meta_lo = meta_v[pl.ds(0, 16)]
        meta_hi = meta_v[pl.ds(16, 16)]

        def idx_load(g, p):
            pltpu.async_copy(e_hbm.at[c, t, pl.ds(g * GRAN, GRAN)],
                             ering[p], isem[p])
            pltpu.async_copy(loc_hbm.at[c, t, pl.ds(g * GRAN, GRAN)],
                             lring[p], isem[p])

        def idx_wait(p):
            pltpu.make_async_copy(e_hbm.at[0, 0, pl.ds(0, GRAN)],
                                  ering[p], isem[p]).wait()
            pltpu.make_async_copy(loc_hbm.at[0, 0, pl.ds(0, GRAN)],
                                  lring[p], isem[p]).wait()

        def accumulate(s):
            zcopies = [
                pltpu.async_copy(zero_v, acc.at[pl.ds(t * slab + i * 8, 8)],
                                 gsem[i % NRING])
                for i in range(slab // 8)]
            for cp in zcopies:
                cp.wait()

            plsc.subcore_barrier()
            g0 = jnp.sum(jnp.where(iota == s, meta_lo, 0)) >> 4
            g1 = jnp.sum(jnp.where(iota == s, meta_hi, 0)) >> 4

            for p in range(NRING):
                @pl.when(g0 + p < g1)
                def _():
                    idx_load(g0 + p, p)

            def add_wait(p):
                pltpu.make_async_copy(buf[p], acc.at[lring[p]],
                                      asem[p]).wait()

            def round_body(rr, _):
                g = g0 + rr * NRING
                for p in range(NRING):
                    idx_wait(p)

                    @pl.when(rr > 0)
                    def _():
                        add_wait(p)

                    pltpu.async_copy(t_hbm.at[ering[p]], buf[p], gsem[p])
                for p in range(NRING):
                    pltpu.make_async_copy(t_hbm.at[ering[p]], buf[p],
                                          gsem[p]).wait()
                    pltpu.async_copy(buf[p], acc.at[lring[p]], asem[p],
                                     add=True)
                    nxt = g + p + NRING

                    @pl.when(nxt < g1)
                    def _():
                        idx_load(nxt, p)
                return 0

            rounds = (g1 - g0) // NRING
            lax.fori_loop(0, rounds, round_body, 0)
            for p in range(NRING):
                @pl.when(rounds > 0)
                def _():
                    add_wait(p)
            plsc.subcore_barrier()

        def bucket_body(s, _):
            accumulate(s)
            b = 2 * s + c
            pltpu.sync_copy(
                acc.at[pl.ds(t * slab, slab)],
                out_hbm.at[pl.ds(b * BROWS + t * slab, slab)])
            return 0

        lax.fori_loop(0, (NBKT - 1) // 2, bucket_body, 0)

        tail_rows = N - (NBKT - 1) * BROWS
        full = tail_rows // slab
        rem = tail_rows % slab

        @pl.when(c == (NBKT - 1) % 2)
        def _():
            accumulate(jnp.int32((NBKT - 1) // 2))

            @pl.when(t < full)
            def _():
                pltpu.sync_copy(
                    acc.at[pl.ds(t * slab, slab)],
                    out_hbm.at[pl.ds((NBKT - 1) * BROWS + t * slab, slab)])

            if rem:

                @pl.when(t == full)
                def _():
                    pltpu.sync_copy(
                        acc.at[pl.ds(full * slab, rem)],
                        out_hbm.at[
                            pl.ds((NBKT - 1) * BROWS + full * slab, rem)])

    return k(transformed, e_out, loc_out, meta_out)


def kernel(features, nbmap, coords, kernel):
    src = nbmap[:, :, 0].reshape(-1)
    src_pad = jnp.concatenate([src, jnp.zeros((E_PAD - E,), jnp.int32)])
    dst = nbmap[:, :, 1].reshape(-1)
    dst_pad = jnp.concatenate(
        [dst, jnp.full((E_PAD - E,), 1 << 20, jnp.int32)])
    e_out, loc_out, meta_out = _sc_scan(dst_pad)
    gathered = _sc_gather(features, src_pad)
    transformed = _tc_matmul(gathered, kernel)
    return _sc_scatter_add(transformed, e_out, loc_out, meta_out)


# speedup vs baseline: 1.3614x; 1.0672x over previous
import dataclasses
import functools

import jax
import jax.numpy as jnp
from jax import lax
from jax.experimental import pallas as pl
from jax.experimental.pallas import tpu as pltpu
from jax.experimental.pallas import tpu_sc as plsc

N = 100000
INC = 128
OUTC = 128
K = 27
EK = 23000
E = K * EK

NC = 2
NS = 16
NW = NC * NS
CHUNK = 256
CHUNKS_PER_W = 76
E_PAD = NW * CHUNK * CHUNKS_PER_W

BLK = 4600
NBLK = E // BLK


def _sc_compiler_params(tc_tiling=True):
    cp = pltpu.CompilerParams()
    fields = pltpu.CompilerParams.__dataclass_fields__
    if "needs_layout_passes" in fields:
        cp = dataclasses.replace(cp, needs_layout_passes=False)
    if not tc_tiling and "use_tc_tiling_on_sc" in fields:
        cp = dataclasses.replace(cp, use_tc_tiling_on_sc=False)
    return cp


def _sc_gather(features, src_pad):
    mesh = plsc.VectorSubcoreMesh(core_axis_name="c", subcore_axis_name="s")

    @functools.partial(
        pl.kernel,
        out_type=jax.ShapeDtypeStruct((E_PAD, INC), jnp.float32),
        mesh=mesh,
        scratch_types=[
            pltpu.VMEM((CHUNK,), jnp.int32),
            pltpu.VMEM((CHUNK,), jnp.int32),
            pltpu.VMEM((CHUNK, INC), jnp.float32),
            pltpu.VMEM((CHUNK, INC), jnp.float32),
            pltpu.SemaphoreType.DMA,
            pltpu.SemaphoreType.DMA,
            pltpu.SemaphoreType.DMA,
            pltpu.SemaphoreType.DMA,
        ],
    )
    def k(feat_hbm, src_hbm, out_hbm, idx0, idx1, rows0, rows1,
          gsem0, gsem1, wsem0, wsem1):
        wid = lax.axis_index("s") * NC + lax.axis_index("c")
        idx = (idx0, idx1)
        rows = (rows0, rows1)
        gsem = (gsem0, gsem1)
        wsem = (wsem0, wsem1)
        gathers = [None, None]
        writes = [None, None]

        for j in range(CHUNKS_PER_W):
            p = j % 2
            base = (wid * CHUNKS_PER_W + j) * CHUNK
            if writes[p] is not None:
                writes[p].wait()
            pltpu.sync_copy(src_hbm.at[pl.ds(base, CHUNK)], idx[p])
            gathers[p] = pltpu.async_copy(feat_hbm.at[idx[p]], rows[p],
                                          gsem[p])
            q = 1 - p
            if gathers[q] is not None:
                gathers[q].wait()
                prev = base - CHUNK
                writes[q] = pltpu.async_copy(
                    rows[q], out_hbm.at[pl.ds(prev, CHUNK)], wsem[q])
                gathers[q] = None

        last = CHUNKS_PER_W - 1
        p = last % 2
        gathers[p].wait()
        base = (wid * CHUNKS_PER_W + last) * CHUNK
        pltpu.sync_copy(rows[p], out_hbm.at[pl.ds(base, CHUNK)])
        writes[1 - p].wait()

    return k(features, src_pad)


def _tc_matmul(gathered, weights):
    def body(g_ref, w_ref, t_ref):
        t_ref[...] = jnp.dot(g_ref[...], w_ref[0],
                             preferred_element_type=jnp.float32)

    return pl.pallas_call(
        body,
        grid=(NBLK,),
        in_specs=[
            pl.BlockSpec((BLK, INC), lambda i: (i, 0)),
            pl.BlockSpec((1, INC, OUTC), lambda i: (i // (EK // BLK), 0, 0)),
        ],
        out_specs=pl.BlockSpec((BLK, OUTC), lambda i: (i, 0)),
        out_shape=jax.ShapeDtypeStruct((E, OUTC), jnp.float32),
    )(gathered, weights)


NBKT = 25
BROWS = 4096
SHIFT = 12
NSLOT = 13
SLICE = E_PAD // NS
STRIP = 2048
NSTRIP = SLICE // STRIP
VPS = STRIP // 16
GRAN = 32
ROUND = 2 * GRAN
ARENA = -(-(SLICE + NSLOT * (ROUND - 1)) // ROUND) * ROUND
TRASH = BROWS
ACC_ROWS = BROWS + NS * 4


def _sc_scan(dst_pad):
    mesh = plsc.VectorSubcoreMesh(core_axis_name="c", subcore_axis_name="s")

    @functools.partial(
        pl.kernel,
        out_type=(
            jax.ShapeDtypeStruct((NC, NS, ARENA), jnp.int32),
            jax.ShapeDtypeStruct((NC, NS, ARENA), jnp.int32),
            jax.ShapeDtypeStruct((NC, NS, 32), jnp.int32),
        ),
        mesh=mesh,
        compiler_params=_sc_compiler_params(),
        scratch_types=[
            pltpu.VMEM((STRIP,), jnp.int32),
            pltpu.VMEM((ARENA,), jnp.int32),
            pltpu.VMEM((ARENA,), jnp.int32),
            pltpu.VMEM((32,), jnp.int32),
        ],
    )
    def k(dst_hbm, e_out, loc_out, meta_out, dstb_v, e_ar, loc_ar, meta_v):
        c = lax.axis_index("c")
        t = lax.axis_index("s")
        iota = lax.iota(jnp.int32, 16)
        base = t * SLICE

        def strip1(s_i, cnts):
            pltpu.sync_copy(dst_hbm.at[pl.ds(base + s_i * STRIP, STRIP)],
                            dstb_v)

            def vec1(v, cnts):
                d = dstb_v[pl.ds(v * 16, 16)]
                bkt = d >> SHIFT
                return tuple(
                    cnts[s] + jnp.sum(jnp.where(bkt == 2 * s + c, 1, 0))
                    for s in range(NSLOT))

            return lax.fori_loop(0, VPS, vec1, cnts)

        cnts = lax.fori_loop(0, NSTRIP, strip1,
                             tuple(jnp.int32(0) for _ in range(NSLOT)))

        offs = []
        run = jnp.int32(0)
        meta_lo = jnp.zeros((16,), jnp.int32)
        for s in range(NSLOT):
            offs.append(run)
            meta_lo = jnp.where(iota == s, run, meta_lo)
            run = run + ((cnts[s] + ROUND - 1) // ROUND) * ROUND

        def strip2(s_i, curs):
            pltpu.sync_copy(dst_hbm.at[pl.ds(base + s_i * STRIP, STRIP)],
                            dstb_v)

            def vec2(v, curs):
                d = dstb_v[pl.ds(v * 16, 16)]
                bkt = d >> SHIFT
                e_vec = base + s_i * STRIP + v * 16 + iota
                loc_vec = jnp.bitwise_and(d, BROWS - 1)
                new = []
                for s in range(NSLOT):
                    m = bkt == 2 * s + c
                    mi = jnp.where(m, 1, 0)
                    r = plsc.cumsum(mi)
                    pos = curs[s] + r - 1
                    plsc.store_scatter(e_ar, [pos], e_vec, mask=m)
                    plsc.store_scatter(loc_ar, [pos], loc_vec, mask=m)
                    new.append(curs[s] + jnp.sum(mi))
                return tuple(new)

            return lax.fori_loop(0, VPS, vec2, curs)

        curs = lax.fori_loop(0, NSTRIP, strip2, tuple(offs))

        trash = TRASH + t * 4 + jnp.bitwise_and(iota, 3)
        meta_hi = jnp.zeros((16,), jnp.int32)
        for s in range(NSLOT):
            cnt = curs[s] - offs[s]
            end = offs[s] + ((cnt + ROUND - 1) // ROUND) * ROUND
            for kq in range(ROUND // 16):
                pos = curs[s] + kq * 16 + iota
                m = pos < end
                plsc.store_scatter(e_ar, [pos], (kq % 4) * 16 + iota, mask=m)
                plsc.store_scatter(loc_ar, [pos], trash, mask=m)
            meta_hi = jnp.where(iota == s, end, meta_hi)

        meta_v[pl.ds(0, 16)] = meta_lo
        meta_v[pl.ds(16, 16)] = meta_hi
        pltpu.sync_copy(e_ar, e_out.at[c, t])
        pltpu.sync_copy(loc_ar, loc_out.at[c, t])
        pltpu.sync_copy(meta_v, meta_out.at[c, t])

    return k(dst_pad)


def _sc_scatter_add(transformed, e_out, loc_out, meta_out):
    mesh = plsc.VectorSubcoreMesh(core_axis_name="c", subcore_axis_name="s")
    NRING = 2

    @functools.partial(
        pl.kernel,
        out_type=jax.ShapeDtypeStruct((N, OUTC), jnp.float32),
        mesh=mesh,
        compiler_params=_sc_compiler_params(tc_tiling=False),
        scratch_types=(
            [pltpu.VMEM((32,), jnp.int32),
             pltpu.VMEM_SHARED((ACC_ROWS, OUTC), jnp.float32)]
            + [pltpu.VMEM((GRAN,), jnp.int32)] * NRING
            + [pltpu.VMEM((GRAN,), jnp.int32)] * NRING
            + [pltpu.VMEM((GRAN, OUTC), jnp.float32)] * NRING
            + [pltpu.SemaphoreType.DMA] * (3 * NRING)
        ),
    )
    def k(t_hbm, e_hbm, loc_hbm, meta_hbm, out_hbm, meta_v, acc,
          *ring):
        ering = ring[0:NRING]
        lring = ring[NRING:2 * NRING]
        buf = ring[2 * NRING:3 * NRING]
        isem = ring[3 * NRING:4 * NRING]
        gsem = ring[4 * NRING:5 * NRING]
        asem = ring[5 * NRING:6 * NRING]
        c = lax.axis_index("c")
        t = lax.axis_index("s")
        iota = lax.iota(jnp.int32, 16)
        slab = BROWS // NS

        pltpu.sync_copy(meta_hbm.at[c, t], meta_v)
        meta_lo = meta_v[pl.ds(0, 16)]
        meta_hi = meta_v[pl.ds(16, 16)]

        def idx_load(g, p):
            pltpu.async_copy(e_hbm.at[c, t, pl.ds(g * GRAN, GRAN)],
                             ering[p], isem[p])
            pltpu.async_copy(loc_hbm.at[c, t, pl.ds(g * GRAN, GRAN)],
                             lring[p], isem[p])

        def idx_wait(p):
            pltpu.make_async_copy(e_hbm.at[0, 0, pl.ds(0, GRAN)],
                                  ering[p], isem[p]).wait()
            pltpu.make_async_copy(loc_hbm.at[0, 0, pl.ds(0, GRAN)],
                                  lring[p], isem[p]).wait()

        def accumulate(s):
            for r in range(GRAN):
                for q in range(OUTC // 16):
                    buf[0][r, pl.ds(q * 16, 16)] = jnp.zeros(
                        (16,), jnp.float32)
            zcopies = [
                pltpu.async_copy(
                    buf[0], acc.at[pl.ds(t * slab + i * GRAN, GRAN)],
                    gsem[i % NRING])
                for i in range(slab // GRAN)]
            for cp in zcopies:
                cp.wait()

            plsc.subcore_barrier()
            g0 = jnp.sum(jnp.where(iota == s, meta_lo, 0)) >> 5
            g1 = jnp.sum(jnp.where(iota == s, meta_hi, 0)) >> 5

            for p in range(NRING):
                @pl.when(g0 + p < g1)
                def _():
                    idx_load(g0 + p, p)

            def add_wait(p):
                pltpu.make_async_copy(buf[p], acc.at[lring[p]],
                                      asem[p]).wait()

            def round_body(rr, _):
                g = g0 + rr * NRING
                for p in range(NRING):
                    idx_wait(p)

                    @pl.when(rr > 0)
                    def _():
                        add_wait(p)

                    pltpu.async_copy(t_hbm.at[ering[p]], buf[p], gsem[p])
                for p in range(NRING):
                    pltpu.make_async_copy(t_hbm.at[ering[p]], buf[p],
                                          gsem[p]).wait()
                    pltpu.async_copy(buf[p], acc.at[lring[p]], asem[p],
                                     add=True)
                    nxt = g + p + NRING

                    @pl.when(nxt < g1)
                    def _():
                        idx_load(nxt, p)
                return 0

            rounds = (g1 - g0) // NRING
            lax.fori_loop(0, rounds, round_body, 0)
            for p in range(NRING):
                @pl.when(rounds > 0)
                def _():
                    add_wait(p)
            plsc.subcore_barrier()

        def bucket_body(s, _):
            accumulate(s)
            b = 2 * s + c
            pltpu.sync_copy(
                acc.at[pl.ds(t * slab, slab)],
                out_hbm.at[pl.ds(b * BROWS + t * slab, slab)])
            return 0

        lax.fori_loop(0, (NBKT - 1) // 2, bucket_body, 0)

        tail_rows = N - (NBKT - 1) * BROWS
        full = tail_rows // slab
        rem = tail_rows % slab

        @pl.when(c == (NBKT - 1) % 2)
        def _():
            accumulate(jnp.int32((NBKT - 1) // 2))

            @pl.when(t < full)
            def _():
                pltpu.sync_copy(
                    acc.at[pl.ds(t * slab, slab)],
                    out_hbm.at[pl.ds((NBKT - 1) * BROWS + t * slab, slab)])

            if rem:

                @pl.when(t == full)
                def _():
                    pltpu.sync_copy(
                        acc.at[pl.ds(full * slab, rem)],
                        out_hbm.at[
                            pl.ds((NBKT - 1) * BROWS + full * slab, rem)])

    return k(transformed, e_out, loc_out, meta_out)


def kernel(features, nbmap, coords, kernel):
    src = nbmap[:, :, 0].reshape(-1)
    src_pad = jnp.concatenate([src, jnp.zeros((E_PAD - E,), jnp.int32)])
    dst = nbmap[:, :, 1].reshape(-1)
    dst_pad = jnp.concatenate(
        [dst, jnp.full((E_PAD - E,), 1 << 20, jnp.int32)])
    e_out, loc_out, meta_out = _sc_scan(dst_pad)
    gathered = _sc_gather(features, src_pad)
    transformed = _tc_matmul(gathered, kernel)
    return _sc_scatter_add(transformed, e_out, loc_out, meta_out)
